# baseline (speedup 1.0000x reference)
import functools

import jax
import jax.numpy as jnp
import numpy as np
from jax.experimental import pallas as pl
from jax.experimental.pallas import tpu as pltpu

H = 128
W = 128
NG = 1024
RB = 16
CB = 64
NRB = H // RB
NCHUNK = 2
NC = NG // NCHUNK
LOG2E = float(np.log2(np.e))


def _prologue_kernel(dataT_ref, op_ref, drv_ref):
    p = dataT_ref[0]
    x = jnp.tanh(p[0:1])
    y = jnp.tanh(p[1:2])
    xs = 0.5 * (x + 1.0) * W
    ys = 0.5 * (y + 1.0) * H
    sx = jnp.abs(p[2:3]) + 0.3
    sy = jnp.abs(p[3:4]) + 0.3
    theta = jax.nn.sigmoid(p[4:5]) * (2.0 * np.pi)
    cos = jnp.cos(theta)
    sin = jnp.sin(theta)
    sx2 = sx * sx
    sy2 = sy * sy
    sig_a = cos * cos * sx2 + sin * sin * sy2
    sig_b = cos * sin * (sx2 - sy2)
    sig_c = sin * sin * sx2 + cos * cos * sy2
    det = sig_a * sig_c - sig_b * sig_b
    inv_det = 1.0 / det
    aa = (-0.5 * LOG2E) * sig_c * inv_det
    gg = (-0.5 * LOG2E) * sig_a * inv_det
    bb = LOG2E * sig_b * inv_det
    colop = p[5:8] * op_ref[0:1]
    drv_ref[0] = jnp.concatenate([xs, ys, aa, gg, bb, colop], axis=0)


def _raster_kernel(feat_ref, drv_ref, out_ref):
    c = pl.program_id(2)
    jr = pl.program_id(1)

    d = drv_ref[0]
    cy = (jr * RB).astype(jnp.float32) + (RB // 2)
    yt = d[1:2] - cy
    aa = d[2:3]
    gg = d[3:4]
    bb = d[4:5]
    colop8 = jnp.concatenate(
        [d[5:8], jnp.zeros((5, NC), jnp.float32)],
        axis=0).astype(jnp.bfloat16).T

    ggyt = gg * yt
    bbyt = bb * yt
    kv_base = -2.0 * ggyt
    k_y = ggyt * yt

    @pl.when(c == 0)
    def _():
        out_ref[...] = jnp.zeros_like(out_ref)

    for half in range(2):
        cx = float(half * CB + CB // 2)
        xt = d[0:1] - cx
        aaxt = aa * xt
        k0 = aaxt * xt + k_y + (bb * xt) * yt
        ku = -2.0 * aaxt - bbyt
        kv = kv_base - bb * xt
        kmat = jnp.concatenate(
            [k0, ku, kv, aa, gg, bb, jnp.zeros((2, NC), jnp.float32)],
            axis=0)

        power2 = jax.lax.dot_general(
            feat_ref[...], kmat, (((1,), (0,)), ((), ())),
            precision=jax.lax.Precision.HIGHEST,
            preferred_element_type=jnp.float32)
        alpha = jnp.exp2(jnp.maximum(power2, -280.0)).astype(jnp.bfloat16)

        res = jax.lax.dot_general(
            alpha, colop8, (((1,), (0,)), ((), ())),
            preferred_element_type=jnp.float32)
        contrib = res.T.reshape(8, RB, CB)
        out_ref[0, :, :, half * CB:(half + 1) * CB] += contrib


@functools.partial(jax.jit, static_argnames=())
def kernel(data, opacity, background):
    bsz = data.shape[0]
    dataT = data.transpose(0, 2, 1)
    opT = opacity.reshape(1, NG)

    vv, uu = jnp.meshgrid(
        jnp.arange(RB, dtype=jnp.float32) - (RB // 2 - 0.5),
        jnp.arange(CB, dtype=jnp.float32) - (CB // 2 - 0.5),
        indexing="ij")
    u = uu.reshape(-1)
    v = vv.reshape(-1)
    feat = jnp.stack(
        [jnp.ones_like(u), u, v, u * u, v * v, u * v,
         jnp.zeros_like(u), jnp.zeros_like(u)], axis=1)

    derived = pl.pallas_call(
        _prologue_kernel,
        grid=(bsz,),
        in_specs=[
            pl.BlockSpec((1, 8, NG), lambda b: (b, 0, 0)),
            pl.BlockSpec((1, NG), lambda b: (0, 0)),
        ],
        out_specs=pl.BlockSpec((1, 8, NG), lambda b: (b, 0, 0)),
        out_shape=jax.ShapeDtypeStruct((bsz, 8, NG), jnp.float32),
    )(dataT, opT)

    out_pal = pl.pallas_call(
        _raster_kernel,
        grid=(bsz, NRB, NCHUNK),
        in_specs=[
            pl.BlockSpec((RB * CB, 8), lambda b, jr, c: (0, 0)),
            pl.BlockSpec((1, 8, NC), lambda b, jr, c: (b, 0, c)),
        ],
        out_specs=pl.BlockSpec(
            (1, 8, RB, W), lambda b, jr, c: (b, 0, jr, 0)),
        out_shape=jax.ShapeDtypeStruct((bsz, 8, H, W), jnp.float32),
        compiler_params=pltpu.CompilerParams(
            dimension_semantics=("parallel", "parallel", "arbitrary")),
    )(feat, derived)

    return out_pal[:, :3] + background[None, :, None, None]


# speedup vs baseline: 2.6148x; 2.6148x over previous
import functools

import jax
import jax.numpy as jnp
import numpy as np
from jax.experimental import pallas as pl
from jax.experimental.pallas import tpu as pltpu

H = 128
W = 128
NG = 1024
RB = 16
CB = 64
NRB = H // RB
NCHUNK = 2
NC = NG // NCHUNK
LOG2E = float(np.log2(np.e))


def _prologue_kernel(dataT_ref, op_ref, drv_ref):
    p = dataT_ref[0]
    x = jnp.tanh(p[0:1])
    y = jnp.tanh(p[1:2])
    xs = 0.5 * (x + 1.0) * W
    ys = 0.5 * (y + 1.0) * H
    sx = jnp.abs(p[2:3]) + 0.3
    sy = jnp.abs(p[3:4]) + 0.3
    theta = jax.nn.sigmoid(p[4:5]) * (2.0 * np.pi)
    cos = jnp.cos(theta)
    sin = jnp.sin(theta)
    sx2 = sx * sx
    sy2 = sy * sy
    sig_a = cos * cos * sx2 + sin * sin * sy2
    sig_b = cos * sin * (sx2 - sy2)
    sig_c = sin * sin * sx2 + cos * cos * sy2
    det = sig_a * sig_c - sig_b * sig_b
    inv_det = 1.0 / det
    aa = (-0.5 * LOG2E) * sig_c * inv_det
    gg = (-0.5 * LOG2E) * sig_a * inv_det
    bb = LOG2E * sig_b * inv_det
    colop = p[5:8] * op_ref[0:1]
    drv_ref[0] = jnp.concatenate([xs, ys, aa, gg, bb, colop], axis=0)


def _split3(k8):
    hi = k8.astype(jnp.bfloat16)
    r1 = k8 - hi.astype(jnp.float32)
    mid = r1.astype(jnp.bfloat16)
    lo = (r1 - mid.astype(jnp.float32)).astype(jnp.bfloat16)
    return jnp.concatenate([hi, mid, lo], axis=0)


def _raster_kernel(feat_ref, drv_ref, out_ref):
    c = pl.program_id(2)
    jr = pl.program_id(1)

    d = drv_ref[0]
    cy = (jr * RB).astype(jnp.float32) + (RB // 2)
    yt = d[1:2] - cy
    aa = d[2:3]
    gg = d[3:4]
    bb = d[4:5]
    colop8 = jnp.concatenate(
        [d[5:8], jnp.zeros((5, NC), jnp.float32)],
        axis=0).astype(jnp.bfloat16).T

    ggyt = gg * yt
    bbyt = bb * yt
    kv_base = -2.0 * ggyt
    k_y = ggyt * yt

    @pl.when(c == 0)
    def _():
        out_ref[...] = jnp.zeros_like(out_ref)

    for half in range(2):
        cx = float(half * CB + CB // 2)
        xt = d[0:1] - cx
        aaxt = aa * xt
        k0 = aaxt * xt + k_y + (bb * xt) * yt
        ku = -2.0 * aaxt - bbyt
        kv = kv_base - bb * xt
        k8 = jnp.concatenate([k0, ku, kv, gg, aa, aa, bb, bb], axis=0)
        k24 = _split3(k8)

        power2 = jax.lax.dot_general(
            feat_ref[...], k24, (((1,), (0,)), ((), ())),
            preferred_element_type=jnp.float32)
        alpha = jnp.exp2(jnp.maximum(power2, -280.0)).astype(jnp.bfloat16)

        res = jax.lax.dot_general(
            alpha, colop8, (((1,), (0,)), ((), ())),
            preferred_element_type=jnp.float32)
        contrib = res.T.reshape(8, RB, CB)
        out_ref[0, :, :, half * CB:(half + 1) * CB] += contrib


@functools.partial(jax.jit, static_argnames=())
def kernel(data, opacity, background):
    bsz = data.shape[0]
    dataT = data.transpose(0, 2, 1)
    opT = opacity.reshape(1, NG)

    vv, uu = jnp.meshgrid(
        jnp.arange(RB, dtype=jnp.float32) - (RB // 2 - 0.5),
        jnp.arange(CB, dtype=jnp.float32) - (CB // 2 - 0.5),
        indexing="ij")
    u = uu.reshape(-1)
    v = vv.reshape(-1)

    def hilo(x):
        hi = x.astype(jnp.bfloat16).astype(jnp.float32)
        return hi, x - hi

    u2h, u2l = hilo(u * u)
    uvh, uvl = hilo(u * v)
    f8 = jnp.stack(
        [jnp.ones_like(u), u, v, v * v, u2h, u2l, uvh, uvl],
        axis=1)
    feat = jnp.concatenate([f8, f8, f8], axis=1).astype(jnp.bfloat16)

    derived = pl.pallas_call(
        _prologue_kernel,
        grid=(bsz,),
        in_specs=[
            pl.BlockSpec((1, 8, NG), lambda b: (b, 0, 0)),
            pl.BlockSpec((1, NG), lambda b: (0, 0)),
        ],
        out_specs=pl.BlockSpec((1, 8, NG), lambda b: (b, 0, 0)),
        out_shape=jax.ShapeDtypeStruct((bsz, 8, NG), jnp.float32),
    )(dataT, opT)

    out_pal = pl.pallas_call(
        _raster_kernel,
        grid=(bsz, NRB, NCHUNK),
        in_specs=[
            pl.BlockSpec((RB * CB, 24), lambda b, jr, c: (0, 0)),
            pl.BlockSpec((1, 8, NC), lambda b, jr, c: (b, 0, c)),
        ],
        out_specs=pl.BlockSpec(
            (1, 8, RB, W), lambda b, jr, c: (b, 0, jr, 0)),
        out_shape=jax.ShapeDtypeStruct((bsz, 8, H, W), jnp.float32),
        compiler_params=pltpu.CompilerParams(
            dimension_semantics=("parallel", "parallel", "arbitrary")),
    )(feat, derived)

    return out_pal[:, :3] + background[None, :, None, None]


# speedup vs baseline: 2.6358x; 1.0080x over previous
import functools

import jax
import jax.numpy as jnp
import numpy as np
from jax.experimental import pallas as pl
from jax.experimental.pallas import tpu as pltpu

H = 128
W = 128
NG = 1024
RB = 16
CB = 64
NRB = H // RB
NCHUNK = 2
NC = NG // NCHUNK
LOG2E = float(np.log2(np.e))


def _prologue_kernel(dataT_ref, op_ref, drv_ref):
    p = dataT_ref[0]
    x = jnp.tanh(p[0:1])
    y = jnp.tanh(p[1:2])
    xs = 0.5 * (x + 1.0) * W
    ys = 0.5 * (y + 1.0) * H
    sx = jnp.abs(p[2:3]) + 0.3
    sy = jnp.abs(p[3:4]) + 0.3
    theta = jax.nn.sigmoid(p[4:5]) * (2.0 * np.pi)
    cos = jnp.cos(theta)
    sin = jnp.sin(theta)
    sx2 = sx * sx
    sy2 = sy * sy
    sig_a = cos * cos * sx2 + sin * sin * sy2
    sig_b = cos * sin * (sx2 - sy2)
    sig_c = sin * sin * sx2 + cos * cos * sy2
    det = sig_a * sig_c - sig_b * sig_b
    inv_det = 1.0 / det
    aa = (-0.5 * LOG2E) * sig_c * inv_det
    gg = (-0.5 * LOG2E) * sig_a * inv_det
    bb = LOG2E * sig_b * inv_det
    colop = p[5:8] * op_ref[0:1]
    drv_ref[0] = jnp.concatenate([xs, ys, aa, gg, bb, colop], axis=0)


def _split3(k8):
    hi = k8.astype(jnp.bfloat16)
    r1 = k8 - hi.astype(jnp.float32)
    mid = r1.astype(jnp.bfloat16)
    lo = (r1 - mid.astype(jnp.float32)).astype(jnp.bfloat16)
    zero = jnp.zeros((8, k8.shape[1]), jnp.bfloat16)
    return jnp.concatenate([hi, mid, lo, zero], axis=0)


def _raster_kernel(feat_ref, drv_ref, out_ref):
    c = pl.program_id(2)
    jr = pl.program_id(1)

    d = drv_ref[0]
    cy = (jr * RB).astype(jnp.float32) + (RB // 2)
    yt = d[1:2] - cy
    aa = d[2:3]
    gg = d[3:4]
    bb = d[4:5]
    colop8 = jnp.concatenate(
        [d[5:8], jnp.zeros((5, NC), jnp.float32)],
        axis=0).astype(jnp.bfloat16).T

    ggyt = gg * yt
    bbyt = bb * yt
    kv_base = -2.0 * ggyt
    k_y = ggyt * yt

    @pl.when(c == 0)
    def _():
        out_ref[...] = jnp.zeros_like(out_ref)

    for half in range(2):
        cx = float(half * CB + CB // 2)
        xt = d[0:1] - cx
        aaxt = aa * xt
        k0 = aaxt * xt + k_y + (bb * xt) * yt
        ku = -2.0 * aaxt - bbyt
        kv = kv_base - bb * xt
        k8 = jnp.concatenate([k0, ku, kv, gg, aa, aa, bb, bb], axis=0)
        k32 = _split3(k8)

        power2 = jax.lax.dot_general(
            feat_ref[...], k32, (((1,), (0,)), ((), ())),
            preferred_element_type=jnp.float32)
        alpha = jnp.exp2(jnp.maximum(power2, -280.0)).astype(jnp.bfloat16)

        res = jax.lax.dot_general(
            alpha, colop8, (((1,), (0,)), ((), ())),
            preferred_element_type=jnp.float32)
        contrib = res.T.reshape(8, RB, CB)
        out_ref[0, :, :, half * CB:(half + 1) * CB] += contrib


@functools.partial(jax.jit, static_argnames=())
def kernel(data, opacity, background):
    bsz = data.shape[0]
    dataT = data.transpose(0, 2, 1)
    opT = opacity.reshape(1, NG)

    def np_bf16(x):
        xi = x.astype(np.float32).view(np.uint32)
        r = (xi + np.uint32(0x7FFF) + ((xi >> np.uint32(16)) & np.uint32(1)))
        return (r & np.uint32(0xFFFF0000)).view(np.float32)

    vv, uu = np.meshgrid(
        np.arange(RB, dtype=np.float32) - (RB // 2 - 0.5),
        np.arange(CB, dtype=np.float32) - (CB // 2 - 0.5),
        indexing="ij")
    u = uu.reshape(-1).astype(np.float32)
    v = vv.reshape(-1).astype(np.float32)
    u2h = np_bf16(u * u)
    u2l = (u * u - u2h).astype(np.float32)
    uvh = np_bf16(u * v)
    uvl = (u * v - uvh).astype(np.float32)
    f8 = np.stack(
        [np.ones_like(u), u, v, v * v, u2h, u2l, uvh, uvl],
        axis=1)
    feat_np = np.concatenate([f8, f8, f8, np.zeros_like(f8)], axis=1)
    feat = jnp.asarray(feat_np, dtype=jnp.bfloat16)

    derived = pl.pallas_call(
        _prologue_kernel,
        grid=(bsz,),
        in_specs=[
            pl.BlockSpec((1, 8, NG), lambda b: (b, 0, 0)),
            pl.BlockSpec((1, NG), lambda b: (0, 0)),
        ],
        out_specs=pl.BlockSpec((1, 8, NG), lambda b: (b, 0, 0)),
        out_shape=jax.ShapeDtypeStruct((bsz, 8, NG), jnp.float32),
    )(dataT, opT)

    out_pal = pl.pallas_call(
        _raster_kernel,
        grid=(bsz, NRB, NCHUNK),
        in_specs=[
            pl.BlockSpec((RB * CB, 32), lambda b, jr, c: (0, 0)),
            pl.BlockSpec((1, 8, NC), lambda b, jr, c: (b, 0, c)),
        ],
        out_specs=pl.BlockSpec(
            (1, 8, RB, W), lambda b, jr, c: (b, 0, jr, 0)),
        out_shape=jax.ShapeDtypeStruct((bsz, 8, H, W), jnp.float32),
        compiler_params=pltpu.CompilerParams(
            dimension_semantics=("parallel", "parallel", "arbitrary")),
    )(feat, derived)

    return out_pal[:, :3] + background[None, :, None, None]


# speedup vs baseline: 2.8805x; 1.0928x over previous
import functools

import jax
import jax.numpy as jnp
import numpy as np
from jax.experimental import pallas as pl
from jax.experimental.pallas import tpu as pltpu

H = 128
W = 128
NG = 1024
RB = 8
NC = 512
SC = 128
NROWBLK = H // RB
NCHUNK = NG // NC
NSUB = NC // SC


def _prologue_kernel(dataT_ref, op_ref, drv_ref):
    p = dataT_ref[0]
    x = jnp.tanh(p[0:1])
    y = jnp.tanh(p[1:2])
    xs = 0.5 * (x + 1.0) * W
    ys = 0.5 * (y + 1.0) * H
    sx = jnp.abs(p[2:3]) + 0.3
    sy = jnp.abs(p[3:4]) + 0.3
    theta = jax.nn.sigmoid(p[4:5]) * (2.0 * np.pi)
    cos = jnp.cos(theta)
    sin = jnp.sin(theta)
    sx2 = sx * sx
    sy2 = sy * sy
    sig_a = cos * cos * sx2 + sin * sin * sy2
    sig_b = cos * sin * (sx2 - sy2)
    sig_c = sin * sin * sx2 + cos * cos * sy2
    det = sig_a * sig_c - sig_b * sig_b
    inv_det = 1.0 / det
    ca = (-0.5) * sig_c * inv_det
    cc = (-0.5) * sig_a * inv_det
    cb = -sig_b * inv_det
    colop = p[5:8] * op_ref[0:1]
    drv_ref[0] = jnp.concatenate([xs, ys, ca, cc, cb, colop], axis=0)


def _raster_kernel(drv_ref, out_ref):
    c = pl.program_id(2)
    j = pl.program_id(1)

    d = drv_ref[0]
    yi = (jax.lax.broadcasted_iota(jnp.int32, (RB, 1, 1), 0)
          .astype(jnp.float32) + (j * RB + 0.5).astype(jnp.float32))
    xi = (jax.lax.broadcasted_iota(jnp.int32, (1, W, 1), 1)
          .astype(jnp.float32) + 0.5)

    res = jnp.zeros((RB * W, 8), jnp.float32)
    for s in range(NSUB):
        sl = slice(s * SC, (s + 1) * SC)
        xs = d[0:1, sl].reshape(1, 1, SC)
        ys = d[1:2, sl].reshape(1, 1, SC)
        ca = d[2:3, sl].reshape(1, 1, SC)
        cc = d[3:4, sl].reshape(1, 1, SC)
        cb = d[4:5, sl].reshape(1, 1, SC)
        dx = xi - xs
        dy = yi - ys
        tx = ca * dx * dx
        ty = cc * dy * dy
        dxb = cb * dx
        power = (tx + ty) - dxb * dy
        alpha = jnp.exp(power).astype(jnp.bfloat16).reshape(RB * W, SC)

        colop8 = jnp.concatenate(
            [d[5:8, sl], jnp.zeros((5, SC), jnp.float32)],
            axis=0).astype(jnp.bfloat16)
        res = res + jax.lax.dot_general(
            alpha, colop8.T, (((1,), (0,)), ((), ())),
            preferred_element_type=jnp.float32)

    contrib = res.T.reshape(8, RB, W)

    @pl.when(c == 0)
    def _():
        out_ref[...] = jnp.zeros_like(out_ref)
    out_ref[0] += contrib


@functools.partial(jax.jit, static_argnames=())
def kernel(data, opacity, background):
    bsz = data.shape[0]
    dataT = data.transpose(0, 2, 1)
    opT = opacity.reshape(1, NG)

    derived = pl.pallas_call(
        _prologue_kernel,
        grid=(bsz,),
        in_specs=[
            pl.BlockSpec((1, 8, NG), lambda b: (b, 0, 0)),
            pl.BlockSpec((1, NG), lambda b: (0, 0)),
        ],
        out_specs=pl.BlockSpec((1, 8, NG), lambda b: (b, 0, 0)),
        out_shape=jax.ShapeDtypeStruct((bsz, 8, NG), jnp.float32),
    )(dataT, opT)

    out_pal = pl.pallas_call(
        _raster_kernel,
        grid=(bsz, NROWBLK, NCHUNK),
        in_specs=[
            pl.BlockSpec((1, 8, NC), lambda b, j, c: (b, 0, c)),
        ],
        out_specs=pl.BlockSpec((1, 8, RB, W), lambda b, j, c: (b, 0, j, 0)),
        out_shape=jax.ShapeDtypeStruct((bsz, 8, H, W), jnp.float32),
        compiler_params=pltpu.CompilerParams(
            dimension_semantics=("parallel", "parallel", "arbitrary")),
    )(derived)

    return out_pal[:, :3] + background[None, :, None, None]


# speedup vs baseline: 4.2479x; 1.4747x over previous
import functools

import jax
import jax.numpy as jnp
import numpy as np
from jax.experimental import pallas as pl
from jax.experimental.pallas import tpu as pltpu

H = 128
W = 128
NG = 1024
RB = 16
NC = 1024
SC = 128
NROWBLK = H // RB
NCHUNK = NG // NC
NSUB = NC // SC


def _prologue_kernel(dataT_ref, op_ref, drv_ref):
    p = dataT_ref[0]
    x = jnp.tanh(p[0:1])
    y = jnp.tanh(p[1:2])
    xs = 0.5 * (x + 1.0) * W
    ys = 0.5 * (y + 1.0) * H
    sx = jnp.abs(p[2:3]) + 0.3
    sy = jnp.abs(p[3:4]) + 0.3
    theta = jax.nn.sigmoid(p[4:5]) * (2.0 * np.pi)
    cos = jnp.cos(theta)
    sin = jnp.sin(theta)
    sx2 = sx * sx
    sy2 = sy * sy
    sig_a = cos * cos * sx2 + sin * sin * sy2
    sig_b = cos * sin * (sx2 - sy2)
    sig_c = sin * sin * sx2 + cos * cos * sy2
    det = sig_a * sig_c - sig_b * sig_b
    inv_det = 1.0 / det
    ca = (-0.5) * sig_c * inv_det
    cc = (-0.5) * sig_a * inv_det
    cb = -sig_b * inv_det
    colop = p[5:8] * op_ref[0:1]
    drv_ref[0] = jnp.concatenate([xs, ys, ca, cc, cb, colop], axis=0)


def _raster_kernel(drv_ref, out_ref):
    j = pl.program_id(1)

    d = drv_ref[0]
    yi = (jax.lax.broadcasted_iota(jnp.int32, (RB, 1, 1), 0)
          .astype(jnp.float32) + (j * RB + 0.5).astype(jnp.float32))
    xi = (jax.lax.broadcasted_iota(jnp.int32, (1, W, 1), 1)
          .astype(jnp.float32) + 0.5)

    res = jnp.zeros((RB * W, 8), jnp.float32)
    for s in range(NSUB):
        sl = slice(s * SC, (s + 1) * SC)
        xs = d[0:1, sl].reshape(1, 1, SC)
        ys = d[1:2, sl].reshape(1, 1, SC)
        ca = d[2:3, sl].reshape(1, 1, SC)
        cc = d[3:4, sl].reshape(1, 1, SC)
        cb = d[4:5, sl].reshape(1, 1, SC)
        dx = xi - xs
        dy = yi - ys
        tx = ca * dx * dx
        ty = cc * dy * dy
        dxb = cb * dx
        power = (tx + ty) - dxb * dy
        alpha = jnp.exp(power).astype(jnp.bfloat16).reshape(RB * W, SC)

        colop8 = jnp.concatenate(
            [d[5:8, sl], jnp.zeros((5, SC), jnp.float32)],
            axis=0).astype(jnp.bfloat16)
        res = res + jax.lax.dot_general(
            alpha, colop8.T, (((1,), (0,)), ((), ())),
            preferred_element_type=jnp.float32)

    out_ref[0] = res.T.reshape(8, RB, W)


@functools.partial(jax.jit, static_argnames=())
def kernel(data, opacity, background):
    bsz = data.shape[0]
    dataT = data.transpose(0, 2, 1)
    opT = opacity.reshape(1, NG)

    derived = pl.pallas_call(
        _prologue_kernel,
        grid=(bsz,),
        in_specs=[
            pl.BlockSpec((1, 8, NG), lambda b: (b, 0, 0)),
            pl.BlockSpec((1, NG), lambda b: (0, 0)),
        ],
        out_specs=pl.BlockSpec((1, 8, NG), lambda b: (b, 0, 0)),
        out_shape=jax.ShapeDtypeStruct((bsz, 8, NG), jnp.float32),
    )(dataT, opT)

    out_pal = pl.pallas_call(
        _raster_kernel,
        grid=(bsz, NROWBLK),
        in_specs=[
            pl.BlockSpec((1, 8, NC), lambda b, j: (b, 0, 0)),
        ],
        out_specs=pl.BlockSpec((1, 8, RB, W), lambda b, j: (b, 0, j, 0)),
        out_shape=jax.ShapeDtypeStruct((bsz, 8, H, W), jnp.float32),
        compiler_params=pltpu.CompilerParams(
            dimension_semantics=("parallel", "parallel")),
    )(derived)

    return out_pal[:, :3] + background[None, :, None, None]
